# Initial kernel scaffold; baseline (speedup 1.0000x reference)
import jax
import jax.numpy as jnp
from jax.experimental import pallas as pl


def kernel(x):
    raise NotImplementedError("write your pallas kernel here")


# speedup vs baseline: 24.2814x; 24.2814x over previous
import functools

import jax
import jax.numpy as jnp
from jax import lax
from jax.experimental import pallas as pl
from jax.experimental.pallas import tpu as pltpu

_K = 512
_N = 1_000_000
_ROWS = 8192
_PAD = _ROWS * 128


def _topk_mask_kernel(x_ref, o_ref, keys_ref):
    x = x_ref[...]
    u = lax.bitcast_convert_type(x, jnp.uint32)
    sgn = u >> jnp.uint32(31)
    keys = jnp.where(sgn == jnp.uint32(1), ~u, u | jnp.uint32(0x80000000))
    keys_ref[...] = keys

    def val_round(r, t):
        cand = t | (jnp.uint32(1) << (jnp.uint32(31) - jnp.uint32(r)))
        cnt = jnp.sum((keys_ref[...] >= cand).astype(jnp.int32))
        return jnp.where(cnt >= _K, cand, t)

    T = lax.fori_loop(0, 32, val_round, jnp.uint32(0))

    eq = keys_ref[...] == T
    n_gt = jnp.sum((keys_ref[...] > T).astype(jnp.int32))
    need = _K - n_gt

    row = lax.broadcasted_iota(jnp.int32, (_ROWS, 128), 0)
    col = lax.broadcasted_iota(jnp.int32, (_ROWS, 128), 1)
    idx = row * 128 + col

    def idx_round(r, t2):
        cand = t2 | (jnp.int32(1) << (jnp.int32(19) - jnp.int32(r)))
        cnt = jnp.sum((eq & (idx < cand)).astype(jnp.int32))
        return jnp.where(cnt < need, cand, t2)

    n_eq = jnp.sum(eq.astype(jnp.int32))
    t2 = lax.cond(
        n_eq == need,
        lambda: jnp.int32(_PAD),
        lambda: lax.fori_loop(0, 20, idx_round, jnp.int32(0)),
    )

    keep = (keys_ref[...] > T) | (eq & (idx <= t2))
    o_ref[...] = jnp.where(keep, x, jnp.float32(0.0))


@functools.partial(jax.jit)
def kernel(x):
    xp = jnp.concatenate(
        [x, jnp.full((_PAD - _N,), -jnp.inf, dtype=jnp.float32)]
    ).reshape(_ROWS, 128)
    out = pl.pallas_call(
        _topk_mask_kernel,
        out_shape=jax.ShapeDtypeStruct((_ROWS, 128), jnp.float32),
        scratch_shapes=[pltpu.VMEM((_ROWS, 128), jnp.uint32)],
    )(xp)
    return out.reshape(-1)[:_N]
